# baseline (device time: 17981 ns/iter reference)
import jax
import jax.numpy as jnp
from jax import lax
from jax.experimental import pallas as pl
from jax.experimental.pallas import tpu as pltpu

N_DEV = 8
M, N = 1024, 512
CH = M // N_DEV
SEND_ORDER = (6, 2, 5, 7, 1, 3, 4)
RECV_ORDER = (1, 3, 4, 2, 5, 7, 6)
C = 2
HN = N // C


def kernel(x):
    m, n = x.shape
    assert (m, n) == (M, N)

    def body(x_ref, out_ref, xq_ref, rs_ref, sc1_ref, sc2a_ref, sc2b_ref,
             send1, recv1, send2, recv2, ssend1, srecv1, ssend2, srecv2):
        pos = lax.axis_index("i")
        sc2 = [sc2a_ref, sc2b_ref]

        s1 = jnp.maximum(jnp.max(jnp.abs(x_ref[...])), 1e-6)
        sc1_ref[0, :] = jnp.full((128,), s1, jnp.float32)
        xq_ref[...] = jnp.rint(x_ref[...] * (127.0 / s1)).astype(jnp.int8)

        barrier = pltpu.get_barrier_semaphore()
        for k in range(1, N_DEV):
            pl.semaphore_signal(
                barrier, inc=1,
                device_id=(pos ^ k,),
                device_id_type=pl.DeviceIdType.MESH,
            )
        pl.semaphore_wait(barrier, N_DEV - 1)

        drain = []

        for k in SEND_ORDER:
            dst = pos ^ k
            sc = pltpu.make_async_remote_copy(
                src_ref=sc1_ref.at[0],
                dst_ref=sc1_ref.at[k],
                send_sem=ssend1.at[k],
                recv_sem=srecv1.at[k],
                device_id=(dst,),
                device_id_type=pl.DeviceIdType.MESH,
            )
            sc.start()
            drain.append(sc)
        for c in range(C):
            cols = pl.ds(c * HN, HN)
            for k in SEND_ORDER:
                dst = pos ^ k
                rdma = pltpu.make_async_remote_copy(
                    src_ref=xq_ref.at[pl.ds(dst * CH, CH), cols],
                    dst_ref=rs_ref.at[k, :, cols],
                    send_sem=send1.at[k, c],
                    recv_sem=recv1.at[k, c],
                    device_id=(dst,),
                    device_id_type=pl.DeviceIdType.MESH,
                )
                rdma.start()
                drain.append(rdma)

        reds = []
        for c in range(C):
            cols = pl.ds(c * HN, HN)
            red = x_ref[pl.ds(pos * CH, CH), cols]
            for k in RECV_ORDER:
                src = pos ^ k
                if c == 0:
                    screcv = pltpu.make_async_remote_copy(
                        src_ref=sc1_ref.at[0],
                        dst_ref=sc1_ref.at[k],
                        send_sem=ssend1.at[k],
                        recv_sem=srecv1.at[k],
                        device_id=(src,),
                        device_id_type=pl.DeviceIdType.MESH,
                    )
                    screcv.wait_recv()
                recv = pltpu.make_async_remote_copy(
                    src_ref=xq_ref.at[pl.ds(0, CH), cols],
                    dst_ref=rs_ref.at[k, :, cols],
                    send_sem=send1.at[k, c],
                    recv_sem=recv1.at[k, c],
                    device_id=(src,),
                    device_id_type=pl.DeviceIdType.MESH,
                )
                recv.wait_recv()
                red = red + (rs_ref[k, :, c * HN:(c + 1) * HN]
                             .astype(jnp.float32) * (sc1_ref[k, 0] / 127.0))
            reds.append(red)

            s2 = jnp.maximum(jnp.max(jnp.abs(red)), 1e-6)
            sc2[c][0, :] = jnp.full((128,), s2, jnp.float32)
            xq_ref[pl.ds(pos * CH, CH), cols] = (
                jnp.rint(red * (127.0 / s2)).astype(jnp.int8))

            for k in SEND_ORDER:
                dst = pos ^ k
                sc = pltpu.make_async_remote_copy(
                    src_ref=sc2[c].at[0],
                    dst_ref=sc2[c].at[k],
                    send_sem=ssend2.at[k, c],
                    recv_sem=srecv2.at[k, c],
                    device_id=(dst,),
                    device_id_type=pl.DeviceIdType.MESH,
                )
                sc.start()
                drain.append(sc)
                rdma = pltpu.make_async_remote_copy(
                    src_ref=xq_ref.at[pl.ds(pos * CH, CH), cols],
                    dst_ref=xq_ref.at[pl.ds(pos * CH, CH), cols],
                    send_sem=send2.at[k, c],
                    recv_sem=recv2.at[k, c],
                    device_id=(dst,),
                    device_id_type=pl.DeviceIdType.MESH,
                )
                rdma.start()
                drain.append(rdma)

        for c in range(C):
            cols = pl.ds(c * HN, HN)
            out_ref[pl.ds(pos * CH, CH), cols] = reds[c]
            for k in RECV_ORDER:
                src = pos ^ k
                screcv = pltpu.make_async_remote_copy(
                    src_ref=sc2[c].at[0],
                    dst_ref=sc2[c].at[k],
                    send_sem=ssend2.at[k, c],
                    recv_sem=srecv2.at[k, c],
                    device_id=(src,),
                    device_id_type=pl.DeviceIdType.MESH,
                )
                screcv.wait_recv()
                recv = pltpu.make_async_remote_copy(
                    src_ref=xq_ref.at[pl.ds(0, CH), cols],
                    dst_ref=xq_ref.at[pl.ds(src * CH, CH), cols],
                    send_sem=send2.at[k, c],
                    recv_sem=recv2.at[k, c],
                    device_id=(src,),
                    device_id_type=pl.DeviceIdType.MESH,
                )
                recv.wait_recv()
                out_ref[pl.ds(src * CH, CH), cols] = (
                    xq_ref[pl.ds(src * CH, CH), cols].astype(jnp.float32)
                    * (sc2[c][k, 0] / 127.0))

        for rdma in drain:
            rdma.wait_send()

    return pl.pallas_call(
        body,
        out_shape=jax.ShapeDtypeStruct((M, N), jnp.float32),
        in_specs=[pl.BlockSpec(memory_space=pltpu.VMEM)],
        out_specs=pl.BlockSpec(memory_space=pltpu.VMEM),
        scratch_shapes=[
            pltpu.VMEM((M, N), jnp.int8),
            pltpu.VMEM((N_DEV, CH, N), jnp.int8),
            pltpu.VMEM((N_DEV, 128), jnp.float32),
            pltpu.VMEM((N_DEV, 128), jnp.float32),
            pltpu.VMEM((N_DEV, 128), jnp.float32),
            pltpu.SemaphoreType.DMA((N_DEV, C)),
            pltpu.SemaphoreType.DMA((N_DEV, C)),
            pltpu.SemaphoreType.DMA((N_DEV, C)),
            pltpu.SemaphoreType.DMA((N_DEV, C)),
            pltpu.SemaphoreType.DMA((N_DEV,)),
            pltpu.SemaphoreType.DMA((N_DEV,)),
            pltpu.SemaphoreType.DMA((N_DEV, C)),
            pltpu.SemaphoreType.DMA((N_DEV, C)),
        ],
        compiler_params=pltpu.CompilerParams(collective_id=0),
    )(x)
